# baseline (speedup 1.0000x reference)
import jax
import jax.numpy as jnp
import numpy as np
from jax import lax
from jax.experimental import pallas as pl
from jax.experimental.pallas import tpu as pltpu
from jax.experimental.pallas import tpu_sc as plsc

RES = 0.03
NUM_ROTS = 120
GRID = 34
GRID3 = GRID ** 3
C = 39424
NC, NS, L = 2, 16, 16
NPAD = 102400
PTS_PER_TILE = NPAD // NS
SUB = 640
NSUB = PTS_PER_TILE // SUB
GROUPS = SUB // L
QUADS = NUM_ROTS // 4
INV_RES = np.float32(1.0 / RES)
TC_CHUNK = 5632
ROT_TAB = QUADS * L
STG_ROWS = NUM_ROTS * L // 128
STG_W = 128


def _sc_body(px_h, py_h, pz_h, xx_h, xy_h, xz_h, sx_h, sy_h, sz_h, w_h,
             cos_h, sin_h, out_sc, out_acc,
             b0, b1, b2, b3, b4, b5, b6, b7, b8, b9, byb,
             g1, g2, cos_t, sin_t, idx_st, val_st, val1_st, acc, acc1, sem):
    c = lax.axis_index("c")
    s = lax.axis_index("s")
    cf = c.astype(jnp.float32)
    sc0 = jnp.float32(1.0) - cf

    pltpu.sync_copy(cos_h, cos_t)
    pltpu.sync_copy(sin_h, sin_t)

    zero = jnp.zeros((L,), jnp.float32)

    def zbody(j, carry):
        g1[pl.ds(j * L, L)] = zero
        g2[pl.ds(j * L, L)] = zero
        return carry

    lax.fori_loop(0, C // L, zbody, 0)

    @pl.when(s == 0)
    def _():
        pltpu.sync_copy(g1, acc)
        pltpu.sync_copy(g1, acc1)

    plsc.subcore_barrier()

    base_t = s * PTS_PER_TILE

    def fire_row(j):
        pltpu.async_copy(val_st.at[j], acc.at[idx_st.at[j]], sem, add=True)
        pltpu.async_copy(val1_st.at[j], acc1.at[idx_st.at[j]], sem, add=True)

    def wait_row(j):
        pltpu.make_async_copy(val_st.at[j], acc.at[idx_st.at[j]], sem).wait()
        pltpu.make_async_copy(val1_st.at[j], acc1.at[idx_st.at[j]], sem).wait()

    def subchunk(k, carry):
        base = base_t + k * SUB
        pltpu.sync_copy(px_h.at[pl.ds(base, SUB)], b0)
        pltpu.sync_copy(py_h.at[pl.ds(base, SUB)], b1)
        pltpu.sync_copy(pz_h.at[pl.ds(base, SUB)], b2)
        pltpu.sync_copy(xx_h.at[pl.ds(base, SUB)], b3)
        pltpu.sync_copy(xy_h.at[pl.ds(base, SUB)], b4)
        pltpu.sync_copy(xz_h.at[pl.ds(base, SUB)], b5)
        pltpu.sync_copy(sx_h.at[pl.ds(base, SUB)], b6)
        pltpu.sync_copy(sy_h.at[pl.ds(base, SUB)], b7)
        pltpu.sync_copy(sz_h.at[pl.ds(base, SUB)], b8)
        pltpu.sync_copy(w_h.at[pl.ds(base, SUB)], b9)

        def derive(g, carry2):
            o = g * L
            pxv = b0[pl.ds(o, L)]
            pyv = b1[pl.ds(o, L)]
            pzv = b2[pl.ds(o, L)]
            xxv = b3[pl.ds(o, L)]
            xyv = b4[pl.ds(o, L)]
            xzv = b5[pl.ds(o, L)]
            sxv = b6[pl.ds(o, L)]
            syv = b7[pl.ds(o, L)]
            szv = b8[pl.ds(o, L)]
            wv = b9[pl.ds(o, L)]
            pxr = pxv * INV_RES
            pzr = pzv * INV_RES
            oxr = xxv * sxv * INV_RES
            ozr = xzv * szv * INV_RES
            fy = (pyv - xyv * syv) * INV_RES
            iy = jnp.clip(fy, 0.0, 33.0).astype(jnp.int32)
            yb = iy * GRID
            ws = wv * sc0
            a0 = ws + (wv * sxv) * cf
            ab1 = (wv * syv) * cf
            ab2 = (wv * szv) * cf
            b0[pl.ds(o, L)] = pxr
            b2[pl.ds(o, L)] = pzr
            b3[pl.ds(o, L)] = oxr
            b5[pl.ds(o, L)] = ozr
            byb[pl.ds(o, L)] = yb
            b6[pl.ds(o, L)] = a0
            b7[pl.ds(o, L)] = ws
            b8[pl.ds(o, L)] = ab1
            b9[pl.ds(o, L)] = ab2
            return carry2

        lax.fori_loop(0, GROUPS, derive, 0)

        def vote(g, carry2):
            o = g * L
            pxr = b0[pl.ds(o, L)]
            ybi = byb[pl.ds(o, L)]
            pzr = b2[pl.ds(o, L)]
            oxr = b3[pl.ds(o, L)]
            ozr = b5[pl.ds(o, L)]
            a0 = b6[pl.ds(o, L)]
            aa = b7[pl.ds(o, L)]
            ab1 = b8[pl.ds(o, L)]
            ab2 = b9[pl.ds(o, L)]

            gg = k * GROUPS + g
            sbase = (gg % 2) * STG_ROWS

            @pl.when(gg >= 2)
            def _():
                for j in range(STG_ROWS):
                    wait_row(sbase + j)

            def do_votes(cnt, fx, fz, v1, v2):
                ix = jnp.clip(fx, 0.0, 33.0).astype(jnp.int32)
                iz = jnp.clip(fz, 0.0, 33.0).astype(jnp.int32)
                flat = ix * (GRID * GRID) + ybi + iz
                row = sbase + (cnt >> 3)
                col = (cnt & 7) * L
                idx_st[row, pl.ds(col, L)] = flat
                val_st[row, pl.ds(col, L)] = a0
                val1_st[row, pl.ds(col, L)] = v1
                plsc.addupdate_scatter(g2, [flat], v2)

            def rot(r, cr):
                rof, cnt = cr
                crv = cos_t[pl.ds(rof, L)]
                srv = sin_t[pl.ds(rof, L)]
                d1 = crv * oxr - srv * ozr
                d2 = srv * oxr + crv * ozr
                u1 = aa * crv
                u2 = aa * srv
                do_votes(cnt, pxr - d1, pzr - d2, u1 + ab1, u2 + ab2)
                do_votes(cnt + 1, pxr + d2, pzr - d1, ab1 - u2, u1 + ab2)
                do_votes(cnt + 2, pxr + d1, pzr + d2, ab1 - u1, ab2 - u2)
                do_votes(cnt + 3, pxr - d2, pzr + d1, ab1 + u2, ab2 - u1)
                return (rof + L, cnt + 4)

            lax.fori_loop(0, QUADS, rot, (0, 0))

            for j in range(STG_ROWS):
                fire_row(sbase + j)
            return carry2

        lax.fori_loop(0, GROUPS, vote, 0)
        return carry

    lax.fori_loop(0, NSUB, subchunk, 0)

    for j in range(2 * STG_ROWS):
        wait_row(j)

    plsc.subcore_barrier()

    pltpu.sync_copy(g2, out_sc.at[c, s])

    @pl.when(s == 0)
    def _():
        pltpu.sync_copy(acc, out_acc.at[c, 0])
        pltpu.sync_copy(acc1, out_acc.at[c, 1])


_sc_vote = pl.kernel(
    _sc_body,
    out_type=(
        jax.ShapeDtypeStruct((NC, NS, C), jnp.float32),
        jax.ShapeDtypeStruct((NC, 2, C), jnp.float32),
    ),
    mesh=plsc.VectorSubcoreMesh(core_axis_name="c", subcore_axis_name="s"),
    compiler_params=pltpu.CompilerParams(needs_layout_passes=False),
    scratch_types=(
        [pltpu.VMEM((SUB,), jnp.float32)] * 10
        + [pltpu.VMEM((SUB,), jnp.int32)]
        + [pltpu.VMEM((C,), jnp.float32)] * 2
        + [pltpu.VMEM((ROT_TAB,), jnp.float32)] * 2
        + [pltpu.VMEM((2 * STG_ROWS, STG_W), jnp.int32),
           pltpu.VMEM((2 * STG_ROWS, STG_W), jnp.float32),
           pltpu.VMEM((2 * STG_ROWS, STG_W), jnp.float32),
           pltpu.VMEM_SHARED((C,), jnp.float32),
           pltpu.VMEM_SHARED((C,), jnp.float32),
           pltpu.SemaphoreType.DMA]
    ),
)


def _reduce_body(in_ref, out_ref):
    out_ref[...] = jnp.sum(in_ref[...], axis=1)


_tc_reduce = pl.pallas_call(
    _reduce_body,
    grid=(C // TC_CHUNK,),
    in_specs=[pl.BlockSpec((NC, NS, TC_CHUNK), lambda i: (0, 0, i))],
    out_specs=pl.BlockSpec((NC, TC_CHUNK), lambda i: (0, i)),
    out_shape=jax.ShapeDtypeStruct((NC, C), jnp.float32),
)


def kernel(points, xyz, scale, obj):
    n = points.shape[0]
    pad = NPAD - n
    pts = jnp.pad(points, ((0, pad), (0, 0)))
    xy = jnp.pad(xyz, ((0, pad), (0, 0)))
    sc = jnp.pad(scale, ((0, pad), (0, 0)))
    w = jnp.pad(obj, (0, pad))

    theta = np.arange(NUM_ROTS, dtype=np.float32) * np.float32(
        2.0 * np.pi / NUM_ROTS)
    cos_t = np.repeat(np.cos(theta[:QUADS]).astype(np.float32), L)
    sin_t = np.repeat(np.sin(theta[:QUADS]).astype(np.float32), L)

    partials, acc = _sc_vote(
        pts[:, 0], pts[:, 1], pts[:, 2],
        xy[:, 0], xy[:, 1], xy[:, 2],
        sc[:, 0], sc[:, 1], sc[:, 2], w,
        jnp.asarray(cos_t), jnp.asarray(sin_t))

    summed = _tc_reduce(partials)
    grid_obj = acc[0, 0, :GRID3].reshape(GRID, GRID, GRID)
    grid_rot = jnp.stack(
        [acc[0, 1, :GRID3], summed[0, :GRID3]], axis=-1
    ).reshape(GRID, GRID, GRID, 2)
    grid_scale = jnp.stack(
        [acc[1, 0, :GRID3], acc[1, 1, :GRID3], summed[1, :GRID3]], axis=-1
    ).reshape(GRID, GRID, GRID, 3)
    return grid_obj, grid_rot, grid_scale


# speedup vs baseline: 1.1968x; 1.1968x over previous
import jax
import jax.numpy as jnp
import numpy as np
from jax import lax
from jax.experimental import pallas as pl
from jax.experimental.pallas import tpu as pltpu
from jax.experimental.pallas import tpu_sc as plsc

RES = 0.03
NUM_ROTS = 120
GRID = 34
GRID3 = GRID ** 3
C = 39424
NC, NS, L = 2, 16, 16
NPAD = 102400
PTS_PER_TILE = NPAD // NS
SUB = 640
NSUB = PTS_PER_TILE // SUB
GROUPS = SUB // L
QUADS = NUM_ROTS // 4
INV_RES = np.float32(1.0 / RES)
TC_CHUNK = 11264
ROT_TAB = QUADS * L
STG_ROWS = NUM_ROTS * L // 128
STG_W = 128


def _sc_body(px_h, py_h, pz_h, xx_h, xy_h, xz_h, sx_h, sy_h, sz_h, w_h,
             cos_h, sin_h, out_sc, out_acc,
             b0, b1, b2, b3, b4, b5, b6, b7, b8, b9, byb,
             g1, g2, cos_t, sin_t, idx_st, val_st, val1_st, acc, acc1, sem):
    c = lax.axis_index("c")
    s = lax.axis_index("s")
    cf = c.astype(jnp.float32)
    sc0 = jnp.float32(1.0) - cf

    pltpu.sync_copy(cos_h, cos_t)
    pltpu.sync_copy(sin_h, sin_t)

    zero = jnp.zeros((L,), jnp.float32)

    def zbody(j, carry):
        g1[pl.ds(j * L, L)] = zero
        g2[pl.ds(j * L, L)] = zero
        return carry

    lax.fori_loop(0, C // L, zbody, 0)

    @pl.when(s == 0)
    def _():
        pltpu.sync_copy(g1, acc)
        pltpu.sync_copy(g1, acc1)

    plsc.subcore_barrier()

    base_t = s * PTS_PER_TILE

    def fire_row(j):
        pltpu.async_copy(val_st.at[j], acc.at[idx_st.at[j]], sem, add=True)

    def wait_row(j):
        pltpu.make_async_copy(val_st.at[j], acc.at[idx_st.at[j]], sem).wait()

    def fire_row1(j):
        pltpu.async_copy(val1_st.at[j], acc1.at[idx_st.at[j]], sem, add=True)

    def wait_row1(j):
        pltpu.make_async_copy(val1_st.at[j], acc1.at[idx_st.at[j]], sem).wait()

    def subchunk(k, carry):
        base = base_t + k * SUB
        pltpu.sync_copy(px_h.at[pl.ds(base, SUB)], b0)
        pltpu.sync_copy(py_h.at[pl.ds(base, SUB)], b1)
        pltpu.sync_copy(pz_h.at[pl.ds(base, SUB)], b2)
        pltpu.sync_copy(xx_h.at[pl.ds(base, SUB)], b3)
        pltpu.sync_copy(xy_h.at[pl.ds(base, SUB)], b4)
        pltpu.sync_copy(xz_h.at[pl.ds(base, SUB)], b5)
        pltpu.sync_copy(sx_h.at[pl.ds(base, SUB)], b6)
        pltpu.sync_copy(sy_h.at[pl.ds(base, SUB)], b7)
        pltpu.sync_copy(sz_h.at[pl.ds(base, SUB)], b8)
        pltpu.sync_copy(w_h.at[pl.ds(base, SUB)], b9)

        def derive(g, carry2):
            o = g * L
            pxv = b0[pl.ds(o, L)]
            pyv = b1[pl.ds(o, L)]
            pzv = b2[pl.ds(o, L)]
            xxv = b3[pl.ds(o, L)]
            xyv = b4[pl.ds(o, L)]
            xzv = b5[pl.ds(o, L)]
            sxv = b6[pl.ds(o, L)]
            syv = b7[pl.ds(o, L)]
            szv = b8[pl.ds(o, L)]
            wv = b9[pl.ds(o, L)]
            pxr = pxv * INV_RES
            pzr = pzv * INV_RES
            oxr = xxv * sxv * INV_RES
            ozr = xzv * szv * INV_RES
            fy = (pyv - xyv * syv) * INV_RES
            iy = jnp.clip(fy, 0.0, 33.0).astype(jnp.int32)
            yb = iy * GRID
            ws = wv * sc0
            a0 = ws + (wv * sxv) * cf
            ab1 = (wv * syv) * cf
            ab2 = (wv * szv) * cf
            b0[pl.ds(o, L)] = pxr
            b2[pl.ds(o, L)] = pzr
            b3[pl.ds(o, L)] = oxr
            b5[pl.ds(o, L)] = ozr
            byb[pl.ds(o, L)] = yb
            b6[pl.ds(o, L)] = a0
            b7[pl.ds(o, L)] = ws
            b8[pl.ds(o, L)] = ab1
            b9[pl.ds(o, L)] = ab2
            return carry2

        lax.fori_loop(0, GROUPS, derive, 0)

        def run_group(g, sbase, stream_ch1):
            o = g * L
            pxr = b0[pl.ds(o, L)]
            ybi = byb[pl.ds(o, L)]
            pzr = b2[pl.ds(o, L)]
            oxr = b3[pl.ds(o, L)]
            ozr = b5[pl.ds(o, L)]
            a0 = b6[pl.ds(o, L)]
            aa = b7[pl.ds(o, L)]
            ab1 = b8[pl.ds(o, L)]
            ab2 = b9[pl.ds(o, L)]

            def do_votes(cnt, fx, fz, v1, v2):
                ix = jnp.clip(fx, 0.0, 33.0).astype(jnp.int32)
                iz = jnp.clip(fz, 0.0, 33.0).astype(jnp.int32)
                flat = ix * (GRID * GRID) + ybi + iz
                row = sbase + (cnt >> 3)
                col = (cnt & 7) * L
                idx_st[row, pl.ds(col, L)] = flat
                val_st[row, pl.ds(col, L)] = a0
                if stream_ch1:
                    val1_st[row, pl.ds(col, L)] = v1
                else:
                    plsc.addupdate_scatter(g1, [flat], v1)
                plsc.addupdate_scatter(g2, [flat], v2)

            def rot(r, cr):
                rof, cnt = cr
                crv = cos_t[pl.ds(rof, L)]
                srv = sin_t[pl.ds(rof, L)]
                d1 = crv * oxr - srv * ozr
                d2 = srv * oxr + crv * ozr
                u1 = aa * crv
                u2 = aa * srv
                do_votes(cnt, pxr - d1, pzr - d2, u1 + ab1, u2 + ab2)
                do_votes(cnt + 1, pxr + d2, pzr - d1, ab1 - u2, u1 + ab2)
                do_votes(cnt + 2, pxr + d1, pzr + d2, ab1 - u1, ab2 - u2)
                do_votes(cnt + 3, pxr - d2, pzr + d1, ab1 + u2, ab2 - u1)
                return (rof + L, cnt + 4)

            lax.fori_loop(0, QUADS, rot, (0, 0))

        def votepair(gp, carry2):
            pp = k * (GROUPS // 2) + gp

            @pl.when(pp >= 1)
            def _():
                for j in range(STG_ROWS):
                    wait_row(j)
                    wait_row1(j)

            run_group(2 * gp, 0, True)
            for j in range(STG_ROWS):
                fire_row(j)
                fire_row1(j)

            @pl.when(pp >= 1)
            def _():
                for j in range(STG_ROWS):
                    wait_row(STG_ROWS + j)

            run_group(2 * gp + 1, STG_ROWS, False)
            for j in range(STG_ROWS):
                fire_row(STG_ROWS + j)
            return carry2

        lax.fori_loop(0, GROUPS // 2, votepair, 0)
        return carry

    lax.fori_loop(0, NSUB, subchunk, 0)

    for j in range(STG_ROWS):
        wait_row(j)
        wait_row1(j)
        wait_row(STG_ROWS + j)

    plsc.subcore_barrier()

    pltpu.sync_copy(g1, out_sc.at[c, s, pl.ds(0, C)])
    pltpu.sync_copy(g2, out_sc.at[c, s, pl.ds(C, C)])

    @pl.when(s == 0)
    def _():
        pltpu.sync_copy(acc, out_acc.at[c, 0])
        pltpu.sync_copy(acc1, out_acc.at[c, 1])


_sc_vote = pl.kernel(
    _sc_body,
    out_type=(
        jax.ShapeDtypeStruct((NC, NS, 2 * C), jnp.float32),
        jax.ShapeDtypeStruct((NC, 2, C), jnp.float32),
    ),
    mesh=plsc.VectorSubcoreMesh(core_axis_name="c", subcore_axis_name="s"),
    compiler_params=pltpu.CompilerParams(needs_layout_passes=False),
    scratch_types=(
        [pltpu.VMEM((SUB,), jnp.float32)] * 10
        + [pltpu.VMEM((SUB,), jnp.int32)]
        + [pltpu.VMEM((C,), jnp.float32)] * 2
        + [pltpu.VMEM((ROT_TAB,), jnp.float32)] * 2
        + [pltpu.VMEM((2 * STG_ROWS, STG_W), jnp.int32),
           pltpu.VMEM((2 * STG_ROWS, STG_W), jnp.float32),
           pltpu.VMEM((2 * STG_ROWS, STG_W), jnp.float32),
           pltpu.VMEM_SHARED((C,), jnp.float32),
           pltpu.VMEM_SHARED((C,), jnp.float32),
           pltpu.SemaphoreType.DMA]
    ),
)


def _reduce_body(in_ref, acc_ref, out_ref):
    out_ref[...] = jnp.sum(in_ref[...], axis=1) + acc_ref[...]


_tc_reduce = pl.pallas_call(
    _reduce_body,
    grid=(2 * C // TC_CHUNK,),
    in_specs=[pl.BlockSpec((NC, NS, TC_CHUNK), lambda i: (0, 0, i)),
              pl.BlockSpec((NC, TC_CHUNK), lambda i: (0, i))],
    out_specs=pl.BlockSpec((NC, TC_CHUNK), lambda i: (0, i)),
    out_shape=jax.ShapeDtypeStruct((NC, 2 * C), jnp.float32),
)


def kernel(points, xyz, scale, obj):
    n = points.shape[0]
    pad = NPAD - n
    pts = jnp.pad(points, ((0, pad), (0, 0)))
    xy = jnp.pad(xyz, ((0, pad), (0, 0)))
    sc = jnp.pad(scale, ((0, pad), (0, 0)))
    w = jnp.pad(obj, (0, pad))

    theta = np.arange(NUM_ROTS, dtype=np.float32) * np.float32(
        2.0 * np.pi / NUM_ROTS)
    cos_t = np.repeat(np.cos(theta[:QUADS]).astype(np.float32), L)
    sin_t = np.repeat(np.sin(theta[:QUADS]).astype(np.float32), L)

    partials, acc = _sc_vote(
        pts[:, 0], pts[:, 1], pts[:, 2],
        xy[:, 0], xy[:, 1], xy[:, 2],
        sc[:, 0], sc[:, 1], sc[:, 2], w,
        jnp.asarray(cos_t), jnp.asarray(sin_t))

    accpad = jnp.concatenate(
        [acc[:, 1, :], jnp.zeros((NC, C), jnp.float32)], axis=1)
    summed = _tc_reduce(partials, accpad)
    A = summed[0]
    B = summed[1]
    grid_obj = acc[0, 0, :GRID3].reshape(GRID, GRID, GRID)
    grid_rot = jnp.stack(
        [A[:GRID3], A[C:C + GRID3]], axis=-1
    ).reshape(GRID, GRID, GRID, 2)
    grid_scale = jnp.stack(
        [acc[1, 0, :GRID3], B[:GRID3], B[C:C + GRID3]], axis=-1
    ).reshape(GRID, GRID, GRID, 3)
    return grid_obj, grid_rot, grid_scale


# speedup vs baseline: 1.1969x; 1.0001x over previous
import jax
import jax.numpy as jnp
import numpy as np
from jax import lax
from jax.experimental import pallas as pl
from jax.experimental.pallas import tpu as pltpu
from jax.experimental.pallas import tpu_sc as plsc

RES = 0.03
NUM_ROTS = 120
GRID = 34
GRID3 = GRID ** 3
C = 39424
NC, NS, L = 2, 16, 16
NPAD = 102400
PTS_PER_TILE = NPAD // NS
SUB = 640
NSUB = PTS_PER_TILE // SUB
GROUPS = SUB // L
QUADS = NUM_ROTS // 4
INV_RES = np.float32(1.0 / RES)
TC_CHUNK = 11264
ROT_TAB = QUADS * L
STG_ROWS = NUM_ROTS * L // 128
STG_W = 128


def _sc_body(px_h, py_h, pz_h, xx_h, xy_h, xz_h, sx_h, sy_h, sz_h, w_h,
             cos_h, sin_h, out_sc, out_acc,
             b0, b1, b2, b3, b4, b5, b6, b7, b8, b9, byb,
             g1, g2, cos_t, sin_t, idx_st, val_st, val1_st, acc, acc1, sem):
    c = lax.axis_index("c")
    s = lax.axis_index("s")
    cf = c.astype(jnp.float32)
    sc0 = jnp.float32(1.0) - cf

    pltpu.sync_copy(cos_h, cos_t)
    pltpu.sync_copy(sin_h, sin_t)

    zero = jnp.zeros((L,), jnp.float32)

    def zbody(j, carry):
        g1[pl.ds(j * L, L)] = zero
        g2[pl.ds(j * L, L)] = zero
        return carry

    lax.fori_loop(0, C // L, zbody, 0)

    @pl.when(s == 0)
    def _():
        pltpu.sync_copy(g1.at[pl.ds(0, C)], acc)
        pltpu.sync_copy(g1.at[pl.ds(0, C)], acc1)

    plsc.subcore_barrier()

    base_t = s * PTS_PER_TILE

    def fire_row(j):
        pltpu.async_copy(val_st.at[j], acc.at[idx_st.at[j]], sem, add=True)

    def wait_row(j):
        pltpu.make_async_copy(val_st.at[j], acc.at[idx_st.at[j]], sem).wait()

    def fire_row1(j):
        pltpu.async_copy(val1_st.at[j], acc1.at[idx_st.at[j]], sem, add=True)

    def wait_row1(j):
        pltpu.make_async_copy(val1_st.at[j], acc1.at[idx_st.at[j]], sem).wait()

    def subchunk(k, carry):
        base = base_t + k * SUB
        pltpu.sync_copy(px_h.at[pl.ds(base, SUB)], b0)
        pltpu.sync_copy(py_h.at[pl.ds(base, SUB)], b1)
        pltpu.sync_copy(pz_h.at[pl.ds(base, SUB)], b2)
        pltpu.sync_copy(xx_h.at[pl.ds(base, SUB)], b3)
        pltpu.sync_copy(xy_h.at[pl.ds(base, SUB)], b4)
        pltpu.sync_copy(xz_h.at[pl.ds(base, SUB)], b5)
        pltpu.sync_copy(sx_h.at[pl.ds(base, SUB)], b6)
        pltpu.sync_copy(sy_h.at[pl.ds(base, SUB)], b7)
        pltpu.sync_copy(sz_h.at[pl.ds(base, SUB)], b8)
        pltpu.sync_copy(w_h.at[pl.ds(base, SUB)], b9)

        def derive(g, carry2):
            o = g * L
            pxv = b0[pl.ds(o, L)]
            pyv = b1[pl.ds(o, L)]
            pzv = b2[pl.ds(o, L)]
            xxv = b3[pl.ds(o, L)]
            xyv = b4[pl.ds(o, L)]
            xzv = b5[pl.ds(o, L)]
            sxv = b6[pl.ds(o, L)]
            syv = b7[pl.ds(o, L)]
            szv = b8[pl.ds(o, L)]
            wv = b9[pl.ds(o, L)]
            pxr = pxv * INV_RES
            pzr = pzv * INV_RES
            oxr = xxv * sxv * INV_RES
            ozr = xzv * szv * INV_RES
            fy = (pyv - xyv * syv) * INV_RES
            iy = jnp.clip(fy, 0.0, 33.0).astype(jnp.int32)
            yb = iy * GRID
            ws = wv * sc0
            a0 = ws + (wv * sxv) * cf
            ab1 = (wv * syv) * cf
            ab2 = (wv * szv) * cf
            b0[pl.ds(o, L)] = pxr
            b2[pl.ds(o, L)] = pzr
            b3[pl.ds(o, L)] = oxr
            b5[pl.ds(o, L)] = ozr
            byb[pl.ds(o, L)] = yb
            b6[pl.ds(o, L)] = a0
            b7[pl.ds(o, L)] = ws
            b8[pl.ds(o, L)] = ab1
            b9[pl.ds(o, L)] = ab2
            return carry2

        lax.fori_loop(0, GROUPS, derive, 0)

        def run_group(g, sbase, stream_ch1):
            o = g * L
            pxr = b0[pl.ds(o, L)]
            ybi = byb[pl.ds(o, L)]
            pzr = b2[pl.ds(o, L)]
            oxr = b3[pl.ds(o, L)]
            ozr = b5[pl.ds(o, L)]
            a0 = b6[pl.ds(o, L)]
            aa = b7[pl.ds(o, L)]
            ab1 = b8[pl.ds(o, L)]
            ab2 = b9[pl.ds(o, L)]

            def do_votes(cnt, fx, fz, v1, v2):
                ix = jnp.clip(fx, 0.0, 33.0).astype(jnp.int32)
                iz = jnp.clip(fz, 0.0, 33.0).astype(jnp.int32)
                flat = ix * (GRID * GRID) + ybi + iz
                row = sbase + (cnt >> 3)
                col = (cnt & 7) * L
                idx_st[row, pl.ds(col, L)] = flat
                val_st[row, pl.ds(col, L)] = a0
                if stream_ch1:
                    val1_st[row, pl.ds(col, L)] = v1
                else:
                    plsc.addupdate_scatter(g1, [flat], v1)
                plsc.addupdate_scatter(g2, [flat], v2)

            def rot(r, cr):
                rof, cnt = cr
                crv = cos_t[pl.ds(rof, L)]
                srv = sin_t[pl.ds(rof, L)]
                d1 = crv * oxr - srv * ozr
                d2 = srv * oxr + crv * ozr
                u1 = aa * crv
                u2 = aa * srv
                do_votes(cnt, pxr - d1, pzr - d2, u1 + ab1, u2 + ab2)
                do_votes(cnt + 1, pxr + d2, pzr - d1, ab1 - u2, u1 + ab2)
                do_votes(cnt + 2, pxr + d1, pzr + d2, ab1 - u1, ab2 - u2)
                do_votes(cnt + 3, pxr - d2, pzr + d1, ab1 + u2, ab2 - u1)
                return (rof + L, cnt + 4)

            lax.fori_loop(0, QUADS, rot, (0, 0))

        def votepair(gp, carry2):
            pp = k * (GROUPS // 2) + gp

            @pl.when(pp >= 1)
            def _():
                for j in range(STG_ROWS):
                    wait_row(j)
                    wait_row1(j)

            run_group(2 * gp, 0, True)
            for j in range(STG_ROWS):
                fire_row(j)
                fire_row1(j)

            @pl.when(pp >= 1)
            def _():
                for j in range(STG_ROWS):
                    wait_row(STG_ROWS + j)

            run_group(2 * gp + 1, STG_ROWS, False)
            for j in range(STG_ROWS):
                fire_row(STG_ROWS + j)
            return carry2

        lax.fori_loop(0, GROUPS // 2, votepair, 0)
        return carry

    lax.fori_loop(0, NSUB, subchunk, 0)

    for j in range(STG_ROWS):
        wait_row(j)
        wait_row1(j)
        wait_row(STG_ROWS + j)

    plsc.subcore_barrier()

    pltpu.sync_copy(g1.at[pl.ds(0, C)], out_sc.at[c, s, pl.ds(0, C)])
    pltpu.sync_copy(g2, out_sc.at[c, s, pl.ds(C, C)])

    @pl.when(s == 0)
    def _():
        pltpu.sync_copy(acc, out_acc.at[c, 0])
        pltpu.sync_copy(acc1, out_acc.at[c, 1])


_sc_vote = pl.kernel(
    _sc_body,
    out_type=(
        jax.ShapeDtypeStruct((NC, NS, 2 * C), jnp.float32),
        jax.ShapeDtypeStruct((NC, 2, C), jnp.float32),
    ),
    mesh=plsc.VectorSubcoreMesh(core_axis_name="c", subcore_axis_name="s"),
    compiler_params=pltpu.CompilerParams(needs_layout_passes=False),
    scratch_types=(
        [pltpu.VMEM((SUB,), jnp.float32)] * 10
        + [pltpu.VMEM((SUB,), jnp.int32)]
        + [pltpu.VMEM((C + 8,), jnp.float32), pltpu.VMEM((C,), jnp.float32)]
        + [pltpu.VMEM((ROT_TAB,), jnp.float32)] * 2
        + [pltpu.VMEM((2 * STG_ROWS, STG_W), jnp.int32),
           pltpu.VMEM((2 * STG_ROWS, STG_W), jnp.float32),
           pltpu.VMEM((2 * STG_ROWS, STG_W), jnp.float32),
           pltpu.VMEM_SHARED((C,), jnp.float32),
           pltpu.VMEM_SHARED((C,), jnp.float32),
           pltpu.SemaphoreType.DMA]
    ),
)


def _reduce_body(in_ref, acc_ref, out_ref):
    out_ref[...] = jnp.sum(in_ref[...], axis=1) + acc_ref[...]


_tc_reduce = pl.pallas_call(
    _reduce_body,
    grid=(2 * C // TC_CHUNK,),
    in_specs=[pl.BlockSpec((NC, NS, TC_CHUNK), lambda i: (0, 0, i)),
              pl.BlockSpec((NC, TC_CHUNK), lambda i: (0, i))],
    out_specs=pl.BlockSpec((NC, TC_CHUNK), lambda i: (0, i)),
    out_shape=jax.ShapeDtypeStruct((NC, 2 * C), jnp.float32),
)


def kernel(points, xyz, scale, obj):
    n = points.shape[0]
    pad = NPAD - n
    pts = jnp.pad(points, ((0, pad), (0, 0)))
    xy = jnp.pad(xyz, ((0, pad), (0, 0)))
    sc = jnp.pad(scale, ((0, pad), (0, 0)))
    w = jnp.pad(obj, (0, pad))

    theta = np.arange(NUM_ROTS, dtype=np.float32) * np.float32(
        2.0 * np.pi / NUM_ROTS)
    cos_t = np.repeat(np.cos(theta[:QUADS]).astype(np.float32), L)
    sin_t = np.repeat(np.sin(theta[:QUADS]).astype(np.float32), L)

    partials, acc = _sc_vote(
        pts[:, 0], pts[:, 1], pts[:, 2],
        xy[:, 0], xy[:, 1], xy[:, 2],
        sc[:, 0], sc[:, 1], sc[:, 2], w,
        jnp.asarray(cos_t), jnp.asarray(sin_t))

    accpad = jnp.concatenate(
        [acc[:, 1, :], jnp.zeros((NC, C), jnp.float32)], axis=1)
    summed = _tc_reduce(partials, accpad)
    A = summed[0]
    B = summed[1]
    grid_obj = acc[0, 0, :GRID3].reshape(GRID, GRID, GRID)
    grid_rot = jnp.stack(
        [A[:GRID3], A[C:C + GRID3]], axis=-1
    ).reshape(GRID, GRID, GRID, 2)
    grid_scale = jnp.stack(
        [acc[1, 0, :GRID3], B[:GRID3], B[C:C + GRID3]], axis=-1
    ).reshape(GRID, GRID, GRID, 3)
    return grid_obj, grid_rot, grid_scale
